# baseline (speedup 1.0000x reference)
import functools

import jax
import jax.numpy as jnp
from jax import lax
from jax.experimental import pallas as pl
from jax.experimental.pallas import tpu as pltpu
from jax.experimental.pallas import tpu_sc as plsc

BATCH = 16384
DIM = 128
LANES = 16
NUM_CORES = 2
NUM_SUBCORES = 16
NUM_WORKERS = NUM_CORES * NUM_SUBCORES
BPW = BATCH // NUM_WORKERS
CHUNK = 128
NCHUNK = BPW // CHUNK


def _body(uid_hbm, iid_hbm, uemb_hbm, iemb_hbm, out_hbm,
          uid_v, iid_v, u_rows, v_rows, out_v, sem_u, sem_v):
    wid = lax.axis_index("s") * NUM_CORES + lax.axis_index("c")
    base = wid * BPW
    pltpu.sync_copy(uid_hbm.at[pl.ds(base, BPW)], uid_v)
    pltpu.sync_copy(iid_hbm.at[pl.ds(base, BPW)], iid_v)

    lane_iota = lax.iota(jnp.int32, LANES)

    for c in range(NCHUNK):
        cu = pltpu.async_copy(
            uemb_hbm.at[uid_v.at[pl.ds(c * CHUNK, CHUNK)]], u_rows, sem_u)
        cv = pltpu.async_copy(
            iemb_hbm.at[iid_v.at[pl.ds(c * CHUNK, CHUNK)]], v_rows, sem_v)
        cu.wait()
        cv.wait()

        def grp_body(g, carry, c=c):
            s = jnp.zeros((LANES,), jnp.float32)
            for j in range(LANES):
                r = g * LANES + j
                acc = u_rows[r, pl.ds(0, LANES)] * v_rows[r, pl.ds(0, LANES)]
                for k in range(1, DIM // LANES):
                    acc = acc + (u_rows[r, pl.ds(k * LANES, LANES)]
                                 * v_rows[r, pl.ds(k * LANES, LANES)])
                s = jnp.where(lane_iota == j, jnp.sum(acc), s)
            out_v[pl.ds(c * CHUNK + g * LANES, LANES)] = s
            return carry

        lax.fori_loop(0, CHUNK // LANES, grp_body, 0)

    pltpu.sync_copy(out_v, out_hbm.at[pl.ds(base, BPW)])


_tt = functools.partial(
    pl.kernel,
    out_type=jax.ShapeDtypeStruct((BATCH,), jnp.float32),
    mesh=plsc.VectorSubcoreMesh(core_axis_name="c", subcore_axis_name="s"),
    compiler_params=pltpu.CompilerParams(needs_layout_passes=False),
    scratch_types=[
        pltpu.VMEM((BPW,), jnp.int32),
        pltpu.VMEM((BPW,), jnp.int32),
        pltpu.VMEM((CHUNK, DIM), jnp.float32),
        pltpu.VMEM((CHUNK, DIM), jnp.float32),
        pltpu.VMEM((BPW,), jnp.float32),
        pltpu.SemaphoreType.DMA,
        pltpu.SemaphoreType.DMA,
    ],
)(_body)


@jax.jit
def kernel(user_ids, item_ids, user_emb, item_emb):
    return _tt(user_ids.astype(jnp.int32), item_ids.astype(jnp.int32),
               user_emb, item_emb)


# speedup vs baseline: 1.4261x; 1.4261x over previous
import functools

import jax
import jax.numpy as jnp
from jax import lax
from jax.experimental import pallas as pl
from jax.experimental.pallas import tpu as pltpu
from jax.experimental.pallas import tpu_sc as plsc

BATCH = 16384
DIM = 128
LANES = 16
NUM_CORES = 2
NUM_SUBCORES = 16
NUM_WORKERS = NUM_CORES * NUM_SUBCORES
BPW = BATCH // NUM_WORKERS
CHUNK = 128
NCHUNK = BPW // CHUNK
KBLK = DIM // LANES


def _body(uid_hbm, iid_hbm, uemb_hbm, iemb_hbm, out_hbm,
          uid_v, iid_v, u_bufs, v_bufs, tposed, out_v, sems):
    wid = lax.axis_index("s") * NUM_CORES + lax.axis_index("c")
    base = wid * BPW
    pltpu.sync_copy(uid_hbm.at[pl.ds(base, BPW)], uid_v)
    pltpu.sync_copy(iid_hbm.at[pl.ds(base, BPW)], iid_v)

    lane_iota = lax.iota(jnp.int32, LANES)
    scat_base = lane_iota * CHUNK

    def start(c, slot):
        cu = pltpu.async_copy(
            uemb_hbm.at[uid_v.at[pl.ds(c * CHUNK, CHUNK)]], u_bufs.at[slot],
            sems.at[slot, 0])
        cv = pltpu.async_copy(
            iemb_hbm.at[iid_v.at[pl.ds(c * CHUNK, CHUNK)]], v_bufs.at[slot],
            sems.at[slot, 1])
        return cu, cv

    pending = start(0, 0)
    for c in range(NCHUNK):
        slot = c % 2
        if c + 1 < NCHUNK:
            nxt = start(c + 1, 1 - slot)
        pending[0].wait()
        pending[1].wait()
        u_rows = u_bufs.at[slot]
        v_rows = v_bufs.at[slot]

        def row_body(r, carry):
            acc = u_rows[r, pl.ds(0, LANES)] * v_rows[r, pl.ds(0, LANES)]
            for k in range(1, KBLK):
                acc = acc + (u_rows[r, pl.ds(k * LANES, LANES)]
                             * v_rows[r, pl.ds(k * LANES, LANES)])
            plsc.store_scatter(tposed, [scat_base + r], acc)
            return carry

        lax.fori_loop(0, CHUNK, row_body, 0, unroll=4)

        for g in range(CHUNK // LANES):
            s = tposed[pl.ds(g * LANES, LANES)]
            for l in range(1, LANES):
                s = s + tposed[pl.ds(l * CHUNK + g * LANES, LANES)]
            out_v[pl.ds(c * CHUNK + g * LANES, LANES)] = s

        if c + 1 < NCHUNK:
            pending = nxt

    pltpu.sync_copy(out_v, out_hbm.at[pl.ds(base, BPW)])


_tt = functools.partial(
    pl.kernel,
    out_type=jax.ShapeDtypeStruct((BATCH,), jnp.float32),
    mesh=plsc.VectorSubcoreMesh(core_axis_name="c", subcore_axis_name="s"),
    compiler_params=pltpu.CompilerParams(needs_layout_passes=False),
    scratch_types=[
        pltpu.VMEM((BPW,), jnp.int32),
        pltpu.VMEM((BPW,), jnp.int32),
        pltpu.VMEM((2, CHUNK, DIM), jnp.float32),
        pltpu.VMEM((2, CHUNK, DIM), jnp.float32),
        pltpu.VMEM((LANES * CHUNK,), jnp.float32),
        pltpu.VMEM((BPW,), jnp.float32),
        pltpu.SemaphoreType.DMA((2, 2)),
    ],
)(_body)


@jax.jit
def kernel(user_ids, item_ids, user_emb, item_emb):
    return _tt(user_ids.astype(jnp.int32), item_ids.astype(jnp.int32),
               user_emb, item_emb)
